# baseline (speedup 1.0000x reference)
import functools

import jax
import jax.numpy as jnp
from jax import lax
from jax.experimental import pallas as pl
from jax.experimental.pallas import tpu as pltpu
from jax.experimental.pallas import tpu_sc as plsc

_H = 14
_W = 14
_S = _H * _W
_B = 64
_C = 768
_CG = 128
_L = 16
_NW = 32
_NCG = _C // _CG
_UNITS = _B * _NCG
_UPW = _UNITS // _NW


def _sc_body(x_hbm, tp_hbm, masked_hbm, tmpl_hbm, tp_v, x_v, m_v, t_v):
    wid = lax.axis_index("s") * 2 + lax.axis_index("c")
    pltpu.sync_copy(tp_hbm, tp_v)

    def unit_body(u, _):
        uid = u * _NW + wid
        b = uid // _NCG
        c0 = (uid % _NCG) * _CG
        pltpu.sync_copy(x_hbm.at[b, :, :, pl.ds(c0, _CG)], x_v)

        for g in range(_CG // _L):
            gl = g * _L

            mx0 = jnp.full((_L,), -jnp.inf, jnp.float32)
            am0 = jnp.zeros((_L,), jnp.int32)

            @plsc.parallel_loop(0, _H, 1, unroll=2, carry=(mx0, am0))
            def amax_loop(h, carry):
                mx, am = carry
                sbase = h * _W
                nodes = []
                for w in range(0, _W, 2):
                    v0 = x_v[h, w, pl.ds(gl, _L)]
                    v1 = x_v[h, w + 1, pl.ds(gl, _L)]
                    bgt = v1 > v0
                    nodes.append((
                        jnp.where(bgt, v1, v0),
                        jnp.where(bgt, sbase + (w + 1), sbase + w),
                    ))
                while len(nodes) > 1:
                    nxt = []
                    for i in range(0, len(nodes) - 1, 2):
                        (vl, al), (vr, ar) = nodes[i], nodes[i + 1]
                        bgt = vr > vl
                        nxt.append((jnp.where(bgt, vr, vl),
                                    jnp.where(bgt, ar, al)))
                    if len(nodes) % 2:
                        nxt.append(nodes[-1])
                    nodes = nxt
                vrow, arow = nodes[0]
                bc = vrow > mx
                return (jnp.where(bc, vrow, mx), jnp.where(bc, arow, am))

            _, am = amax_loop
            amh = am // _W
            amw = am % _W

            @plsc.parallel_loop(0, _H, 1, unroll=2)
            def out_loop(h):
                hv = jnp.full((_L,), h, jnp.int32)
                for w in range(_W):
                    wv = jnp.full((_L,), w, jnp.int32)
                    tv = plsc.load_gather(tp_v, [amh, amw, hv, wv])
                    xv = x_v[h, w, pl.ds(gl, _L)]
                    t_v[h, w, pl.ds(gl, _L)] = tv
                    m_v[h, w, pl.ds(gl, _L)] = jnp.maximum(xv * tv, 0.0)

        pltpu.sync_copy(t_v, tmpl_hbm.at[b, :, :, pl.ds(c0, _CG)])
        pltpu.sync_copy(m_v, masked_hbm.at[b, :, :, pl.ds(c0, _CG)])
        return 0

    lax.fori_loop(0, _UPW, unit_body, 0)


def kernel(input, t_p):
    x = input
    b, h, w, c = x.shape

    mesh = plsc.VectorSubcoreMesh(core_axis_name="c", subcore_axis_name="s")
    run = functools.partial(
        pl.kernel,
        out_type=[
            jax.ShapeDtypeStruct((b, h, w, c), jnp.float32),
            jax.ShapeDtypeStruct((b, h, w, c), jnp.float32),
        ],
        mesh=mesh,
        compiler_params=pltpu.CompilerParams(
            needs_layout_passes=False, use_tc_tiling_on_sc=False),
        scratch_types=[
            pltpu.VMEM((_H, _W, _H, _W), jnp.float32),
            pltpu.VMEM((_H, _W, _CG), jnp.float32),
            pltpu.VMEM((_H, _W, _CG), jnp.float32),
            pltpu.VMEM((_H, _W, _CG), jnp.float32),
        ],
    )(_sc_body)
    masked, tmpl = run(x, t_p)
    return (masked, x, tmpl)


# speedup vs baseline: 1.8135x; 1.8135x over previous
import functools

import jax
import jax.numpy as jnp
from jax import lax
from jax.experimental import pallas as pl
from jax.experimental.pallas import tpu as pltpu
from jax.experimental.pallas import tpu_sc as plsc

_H = 14
_W = 14
_S = _H * _W
_B = 64
_C = 768
_CG = 128
_L = 16
_NW = 32
_NCG = _C // _CG
_UNITS = _B * _NCG
_UPW = _UNITS // _NW


def _sc_body(x_hbm, tp_hbm, masked_hbm, tmpl_hbm, tp_v, x2, m2, t2, sem):
    wid = lax.axis_index("s") * 2 + lax.axis_index("c")
    pltpu.sync_copy(tp_hbm, tp_v)

    def unit_body(u, _):
        uid = u * _NW + wid
        b = uid // _NCG
        c0 = (uid % _NCG) * _CG
        cps = [
            pltpu.async_copy(
                x_hbm.at[b, hh, :, pl.ds(c0, _CG)],
                x2.at[pl.ds(hh * _W, _W), :], sem)
            for hh in range(_H)
        ]
        for cp in cps:
            cp.wait()

        for g in range(_CG // _L):
            gl = g * _L

            mx0 = jnp.full((_L,), -jnp.inf, jnp.float32)
            am0 = jnp.zeros((_L,), jnp.int32)

            @plsc.parallel_loop(0, _S, 4, unroll=7, carry=(mx0, am0))
            def amax_loop(s, carry):
                mx, am = carry
                v0 = x2[s, pl.ds(gl, _L)]
                v1 = x2[s + 1, pl.ds(gl, _L)]
                v2 = x2[s + 2, pl.ds(gl, _L)]
                v3 = x2[s + 3, pl.ds(gl, _L)]
                b1 = v1 > v0
                m01 = jnp.where(b1, v1, v0)
                a01 = jnp.where(b1, s + 1, s)
                b3 = v3 > v2
                m23 = jnp.where(b3, v3, v2)
                a23 = jnp.where(b3, s + 3, s + 2)
                bb = m23 > m01
                ml = jnp.where(bb, m23, m01)
                al = jnp.where(bb, a23, a01)
                bc = ml > mx
                return (jnp.where(bc, ml, mx), jnp.where(bc, al, am))

            _, am = amax_loop
            base = am * _S

            @plsc.parallel_loop(0, _S, 2, unroll=7)
            def out_loop(s):
                for d in range(2):
                    tv = plsc.load_gather(tp_v, [base + (s + d)])
                    xv = x2[s + d, pl.ds(gl, _L)]
                    t2[s + d, pl.ds(gl, _L)] = tv
                    m2[s + d, pl.ds(gl, _L)] = jnp.maximum(xv * tv, 0.0)

        cps = [
            pltpu.async_copy(
                t2.at[pl.ds(hh * _W, _W), :],
                tmpl_hbm.at[b, hh, :, pl.ds(c0, _CG)], sem)
            for hh in range(_H)
        ] + [
            pltpu.async_copy(
                m2.at[pl.ds(hh * _W, _W), :],
                masked_hbm.at[b, hh, :, pl.ds(c0, _CG)], sem)
            for hh in range(_H)
        ]
        for cp in cps:
            cp.wait()
        return 0

    lax.fori_loop(0, _UPW, unit_body, 0)


def kernel(input, t_p):
    x = input
    b, h, w, c = x.shape

    mesh = plsc.VectorSubcoreMesh(core_axis_name="c", subcore_axis_name="s")
    run = functools.partial(
        pl.kernel,
        out_type=[
            jax.ShapeDtypeStruct((b, h, w, c), jnp.float32),
            jax.ShapeDtypeStruct((b, h, w, c), jnp.float32),
        ],
        mesh=mesh,
        compiler_params=pltpu.CompilerParams(needs_layout_passes=False),
        scratch_types=[
            pltpu.VMEM((_S * _S,), jnp.float32),
            pltpu.VMEM((_S, _CG), jnp.float32),
            pltpu.VMEM((_S, _CG), jnp.float32),
            pltpu.VMEM((_S, _CG), jnp.float32),
            pltpu.SemaphoreType.DMA,
        ],
    )(_sc_body)
    masked, tmpl = run(x, t_p.reshape(h * w * h * w))
    return (masked, x, tmpl)


# speedup vs baseline: 2.2162x; 1.2221x over previous
import jax
import jax.numpy as jnp
from jax.experimental import pallas as pl
from jax.experimental.pallas import tpu as pltpu


def _body(x_ref, tp_ref, masked_ref, tmpl_ref):
    x4 = x_ref[0]
    h, w, c = x4.shape
    s = h * w
    mx = jnp.max(x4, axis=(0, 1))
    ih = jax.lax.broadcasted_iota(jnp.int32, (h, w, c), 0)
    iw = jax.lax.broadcasted_iota(jnp.int32, (h, w, c), 1)
    flat = ih * w + iw
    idx = jnp.min(jnp.where(x4 >= mx[None, None, :], flat, s), axis=(0, 1))
    iota2 = jax.lax.broadcasted_iota(jnp.int32, (s, c), 0)
    onehot = (iota2 == idx[None, :]).astype(jnp.float32)
    tmpl2 = jax.lax.dot_general(
        tp_ref[...], onehot,
        dimension_numbers=(((0,), (0,)), ((), ())),
        preferred_element_type=jnp.float32,
    )
    tmpl4 = tmpl2.reshape(h, w, c)
    tmpl_ref[0] = tmpl4
    masked_ref[0] = jnp.maximum(x4 * tmpl4, 0.0)


def kernel(input, t_p):
    x = input
    b, h, w, c = x.shape
    s = h * w
    tp2 = t_p.reshape(s, s)
    masked, tmpl = pl.pallas_call(
        _body,
        grid=(b,),
        in_specs=[
            pl.BlockSpec((1, h, w, c), lambda i: (i, 0, 0, 0)),
            pl.BlockSpec((s, s), lambda i: (0, 0)),
        ],
        out_specs=[
            pl.BlockSpec((1, h, w, c), lambda i: (i, 0, 0, 0)),
            pl.BlockSpec((1, h, w, c), lambda i: (i, 0, 0, 0)),
        ],
        out_shape=[
            jax.ShapeDtypeStruct((b, h, w, c), jnp.float32),
            jax.ShapeDtypeStruct((b, h, w, c), jnp.float32),
        ],
    )(x, tp2)
    return (masked, x, tmpl)


# speedup vs baseline: 2.5512x; 1.1512x over previous
import jax
import jax.numpy as jnp
from jax.experimental import pallas as pl
from jax.experimental.pallas import tpu as pltpu

_H = 14
_W = 14
_S = _H * _W


def _body(x_ref, tp_ref, masked_ref, tmpl_ref):
    x = x_ref[0]
    s, c = x.shape
    mx = jnp.max(x, axis=0, keepdims=True)
    iota = jax.lax.broadcasted_iota(jnp.int32, (s, c), 0)
    idx = jnp.min(jnp.where(x >= mx, iota, s), axis=0, keepdims=True)
    onehot = (iota == idx).astype(jnp.float32)
    tmpl = jax.lax.dot_general(
        tp_ref[...], onehot,
        dimension_numbers=(((0,), (0,)), ((), ())),
        preferred_element_type=jnp.float32,
    )
    tmpl_ref[0] = tmpl
    masked_ref[0] = jnp.maximum(x * tmpl, 0.0)


def kernel(input, t_p):
    x = input
    b, h, w, c = x.shape
    s = h * w
    x3 = x.reshape(b, s, c)
    tp2 = t_p.reshape(s, s)
    masked, tmpl = pl.pallas_call(
        _body,
        grid=(b,),
        in_specs=[
            pl.BlockSpec((1, s, c), lambda i: (i, 0, 0)),
            pl.BlockSpec((s, s), lambda i: (0, 0)),
        ],
        out_specs=[
            pl.BlockSpec((1, s, c), lambda i: (i, 0, 0)),
            pl.BlockSpec((1, s, c), lambda i: (i, 0, 0)),
        ],
        out_shape=[
            jax.ShapeDtypeStruct((b, s, c), jnp.float32),
            jax.ShapeDtypeStruct((b, s, c), jnp.float32),
        ],
    )(x3, tp2)
    return (masked.reshape(b, h, w, c), x, tmpl.reshape(b, h, w, c))
